# baseline (speedup 1.0000x reference)
import jax
import jax.numpy as jnp
from jax import lax
from jax.experimental import pallas as pl
from jax.experimental.pallas import tpu as pltpu
from jax.experimental.pallas import tpu_sc as plsc

B, L = 4096, 200
DIM = 64
NC, NS = 2, 16
NW = NC * NS
RPW = B // NW
LANES = 16
NCHUNK = L // LANES
GROUP = 4
NGRP = RPW // GROUP
NJ = DIM // LANES


def _any_zero_acc(in_v, b):
    acc = jnp.zeros((LANES,), jnp.int32)
    for c in range(NCHUNK):
        v = in_v[b, pl.ds(c * LANES, LANES)]
        acc = acc | jnp.where(v == 0, 1, 0)
    v = in_v[b, pl.ds(L - LANES, LANES)]
    acc = acc | jnp.where(v == 0, 1, 0)
    return acc


def _scalar_or(acc):
    s = acc[0]
    for i in range(1, LANES):
        s = s | acc[i]
    return s


def _body(in_hbm, tab1_hbm, r0_hbm, out_hbm, in_v, dg_v, r0_v, scr_v, sem, sem_s):
    wid = lax.axis_index("s") * NC + lax.axis_index("c")
    base = wid * RPW

    pltpu.sync_copy(in_hbm.at[pl.ds(base, RPW)], in_v)
    for g in range(GROUP):
        pltpu.sync_copy(tab1_hbm, dg_v.at[g])
    pltpu.sync_copy(r0_hbm, r0_v)

    r0c = [r0_v[pl.ds(j * LANES, LANES)] for j in range(NJ)]

    def compose_and_send(b):
        def fix(c, carry):
            off = pl.multiple_of(c * LANES, LANES)
            v = in_v[b, pl.ds(off, LANES)]
            for lane in range(LANES):
                lrow = off + lane
                sc = jnp.where(v[lane] == 0, 0.0, 1.0).astype(jnp.float32)
                zf = jnp.broadcast_to(sc, (LANES,))
                for j in range(NJ):
                    dc = dg_v[0, lrow, pl.ds(j * LANES, LANES)]
                    scr_v[lrow, pl.ds(j * LANES, LANES)] = r0c[j] + zf * (dc - r0c[j])
            return carry
        lax.fori_loop(0, NCHUNK, fix, 0)
        vt = in_v[b, pl.ds(L - LANES, LANES)]
        for lane in range(L - NCHUNK * LANES, LANES):
            lrow = (L - LANES) + lane
            sc = jnp.where(vt[lane] == 0, 0.0, 1.0).astype(jnp.float32)
            zf = jnp.broadcast_to(sc, (LANES,))
            for j in range(NJ):
                dc = dg_v[0, lrow, pl.ds(j * LANES, LANES)]
                scr_v[lrow, pl.ds(j * LANES, LANES)] = r0c[j] + zf * (dc - r0c[j])
        cp = pltpu.make_async_copy(scr_v, out_hbm.at[base + b], sem_s)
        cp.start()
        cp.wait()

    def group(g, cnt):
        b0 = g * GROUP
        accs = [_any_zero_acc(in_v, b0 + r) for r in range(GROUP)]
        accg = accs[0]
        for r in range(1, GROUP):
            accg = accg | accs[r]
        clean = _scalar_or(accg) == 0

        def fast(c):
            pltpu.make_async_copy(dg_v, out_hbm.at[pl.ds(base + b0, GROUP)],
                                  sem).start()
            return c + GROUP

        def dirty(c):
            def per_row(r, c2):
                b = b0 + r
                rz = _scalar_or(_any_zero_acc(in_v, b))

                def frow(c3):
                    pltpu.make_async_copy(dg_v.at[0], out_hbm.at[base + b],
                                          sem).start()
                    return c3 + 1

                def srow(c3):
                    compose_and_send(b)
                    return c3
                return lax.cond(rz == 0, frow, srow, c2)
            return lax.fori_loop(0, GROUP, per_row, c)

        return lax.cond(clean, fast, dirty, cnt)

    cnt = lax.fori_loop(0, NGRP, group, jnp.int32(0))

    def drain(i, carry):
        pltpu.make_async_copy(dg_v.at[0], out_hbm.at[0], sem).wait()
        return carry

    lax.fori_loop(0, cnt, drain, 0)


def kernel(inputs, pos_encoding):
    inputs = inputs.astype(jnp.int32)
    mesh = plsc.VectorSubcoreMesh(core_axis_name="c", subcore_axis_name="s")
    k = pl.kernel(
        _body,
        out_type=jax.ShapeDtypeStruct((B, L, DIM), jnp.float32),
        mesh=mesh,
        compiler_params=pltpu.CompilerParams(use_tc_tiling_on_sc=False),
        scratch_types=[
            pltpu.VMEM((RPW, L), jnp.int32),
            pltpu.VMEM((GROUP, L, DIM), jnp.float32),
            pltpu.VMEM((DIM,), jnp.float32),
            pltpu.VMEM((L, DIM), jnp.float32),
            pltpu.SemaphoreType.DMA,
            pltpu.SemaphoreType.DMA,
        ],
    )
    return k(inputs, pos_encoding[1:L + 1], pos_encoding[0])


# speedup vs baseline: 1.3560x; 1.3560x over previous
import jax
import jax.numpy as jnp
from jax import lax
from jax.experimental import pallas as pl
from jax.experimental.pallas import tpu as pltpu
from jax.experimental.pallas import tpu_sc as plsc

B, L = 4096, 200
DIM = 64
NC, NS = 2, 16
NW = NC * NS
RPW = B // NW
LANES = 16
NCHUNK = L // LANES
NJ = DIM // LANES


def _body(in_hbm, tab1_hbm, r0_hbm, out_hbm, in_v, d_v, r0_v, scr_v, sem, sem_s):
    wid = lax.axis_index("s") * NC + lax.axis_index("c")
    base = wid * RPW

    pltpu.sync_copy(in_hbm.at[pl.ds(base, RPW)], in_v)
    pltpu.sync_copy(tab1_hbm, d_v)
    pltpu.sync_copy(r0_hbm, r0_v)

    r0c = [r0_v[pl.ds(j * LANES, LANES)] for j in range(NJ)]

    def row(b, fast_cnt):
        acc = jnp.zeros((LANES,), jnp.int32)
        for c in range(NCHUNK):
            v = in_v[b, pl.ds(c * LANES, LANES)]
            acc = acc | jnp.where(v == 0, 1, 0)
        v = in_v[b, pl.ds(L - LANES, LANES)]
        acc = acc | jnp.where(v == 0, 1, 0)
        s = acc[0]
        for i in range(1, LANES):
            s = s | acc[i]
        anyz = s > 0

        def slow(cnt):
            def fix(c, carry):
                off = pl.multiple_of(c * LANES, LANES)
                v = in_v[b, pl.ds(off, LANES)]
                for lane in range(LANES):
                    lrow = off + lane
                    sc = jnp.where(v[lane] == 0, 0.0, 1.0).astype(jnp.float32)
                    zf = jnp.broadcast_to(sc, (LANES,))
                    for j in range(NJ):
                        dc = d_v[lrow, pl.ds(j * LANES, LANES)]
                        scr_v[lrow, pl.ds(j * LANES, LANES)] = r0c[j] + zf * (dc - r0c[j])
                return carry
            lax.fori_loop(0, NCHUNK, fix, 0)
            vt = in_v[b, pl.ds(L - LANES, LANES)]
            for lane in range(L - NCHUNK * LANES, LANES):
                lrow = (L - LANES) + lane
                sc = jnp.where(vt[lane] == 0, 0.0, 1.0).astype(jnp.float32)
                zf = jnp.broadcast_to(sc, (LANES,))
                for j in range(NJ):
                    dc = d_v[lrow, pl.ds(j * LANES, LANES)]
                    scr_v[lrow, pl.ds(j * LANES, LANES)] = r0c[j] + zf * (dc - r0c[j])
            cp = pltpu.make_async_copy(scr_v, out_hbm.at[base + b], sem_s)
            cp.start()
            cp.wait()
            return cnt

        def fast(cnt):
            pltpu.make_async_copy(d_v, out_hbm.at[base + b], sem).start()
            return cnt + 1

        return lax.cond(anyz, slow, fast, fast_cnt)

    fast_cnt = lax.fori_loop(0, RPW, row, jnp.int32(0))

    def drain(i, carry):
        pltpu.make_async_copy(d_v, out_hbm.at[0], sem).wait()
        return carry

    lax.fori_loop(0, fast_cnt, drain, 0)


def kernel(inputs, pos_encoding):
    inputs = inputs.astype(jnp.int32)
    mesh = plsc.VectorSubcoreMesh(core_axis_name="c", subcore_axis_name="s")
    k = pl.kernel(
        _body,
        out_type=jax.ShapeDtypeStruct((B, L, DIM), jnp.float32),
        mesh=mesh,
        scratch_types=[
            pltpu.VMEM((RPW, L), jnp.int32),
            pltpu.VMEM((L, DIM), jnp.float32),
            pltpu.VMEM((DIM,), jnp.float32),
            pltpu.VMEM((L, DIM), jnp.float32),
            pltpu.SemaphoreType.DMA,
            pltpu.SemaphoreType.DMA,
        ],
    )
    return k(inputs, pos_encoding[1:L + 1], pos_encoding[0])
